# baseline (speedup 1.0000x reference)
import functools
import jax
import jax.numpy as jnp
from jax import lax
from jax.experimental import pallas as pl
from jax.experimental.pallas import tpu as pltpu
from jax.experimental.pallas import tpu_sc as plsc

D = 64
B = 16384
H = 50
NC, NS = 2, 16
NW = NC * NS
CHUNK = 128
NUNITS = B * H // CHUNK
PER_W = NUNITS // NW
NBUF = 2
SCALE = 8.0
EG = D // 8


def _sc_embed(x2d, table):
    mesh = plsc.VectorSubcoreMesh(core_axis_name="c", subcore_axis_name="s")

    @functools.partial(
        pl.kernel,
        out_type=jax.ShapeDtypeStruct((H, EG, B // CHUNK, 8, CHUNK),
                                      jnp.float32),
        mesh=mesh,
        scratch_types=[
            pltpu.VMEM((PER_W, CHUNK), jnp.int32),
            pltpu.VMEM((CHUNK, D), jnp.float32),
            pltpu.VMEM((CHUNK, D), jnp.float32),
            pltpu.VMEM((D // 16, CHUNK, 16), jnp.float32),
            pltpu.VMEM((EG, 8, CHUNK), jnp.float32),
            pltpu.VMEM((EG, 8, CHUNK), jnp.float32),
            pltpu.SemaphoreType.DMA,
            pltpu.SemaphoreType.DMA,
            pltpu.SemaphoreType.DMA,
            pltpu.SemaphoreType.DMA,
        ],
        compiler_params=pltpu.CompilerParams(
            use_tc_tiling_on_sc=False, needs_layout_passes=False
        ),
    )
    def k(x_hbm, tab_hbm, out_hbm, idx_v, g0, g1, gp, w0, w1,
          gs0, gs1, ws0, ws1):
        gbuf, wbuf = [g0, g1], [w0, w1]
        gsem, wsem = [gs0, gs1], [ws0, ws1]
        wid = lax.axis_index("s") * NC + lax.axis_index("c")
        u0 = wid * PER_W

        pltpu.sync_copy(x_hbm.at[pl.ds(u0, PER_W)], idx_v)

        for b in range(NBUF):
            pltpu.async_copy(tab_hbm.at[idx_v.at[b]], gbuf[b], gsem[b])

        iota = lax.iota(jnp.int32, 16)

        def transpose_scale(gb, wb):
            def srow(i, c):
                for r in range(8):
                    for t in range(D // 16):
                        gp[t, i * 8 + r, :] = (
                            gb[i * 8 + r, pl.ds(t * 16, 16)] * SCALE
                        )
                return c

            lax.fori_loop(0, CHUNK // 8, srow, 0)

            def gblock(g, c):
                rowv = iota + g * 16
                for eg in range(EG):
                    for es in range(8):
                        e = eg * 8 + es
                        colv = jnp.full((16,), e % 16, jnp.int32)
                        v = plsc.load_gather(gp.at[e // 16], [rowv, colv])
                        wb[eg, es, pl.ds(g * 16, 16)] = v
                return c

            lax.fori_loop(0, CHUNK // 16, gblock, 0)

        def unit_out_copies(u, b, issue):
            h = u // (B // CHUNK)
            bg = lax.rem(u, B // CHUNK)
            for eg in range(EG):
                cp = pltpu.make_async_copy(
                    wbuf[b].at[eg], out_hbm.at[h, eg, bg], wsem[b]
                )
                if issue:
                    cp.start()
                else:
                    cp.wait()

        def outer(g, carry):
            for b in range(NBUF):
                u = g * NBUF + b
                pltpu.make_async_copy(
                    tab_hbm.at[idx_v.at[u]], gbuf[b], gsem[b]
                ).wait()
                @pl.when(u >= NBUF)
                def _():
                    unit_out_copies(u0 + u - NBUF, b, issue=False)

                transpose_scale(gbuf[b], wbuf[b])
                unit_out_copies(u0 + u, b, issue=True)

                @pl.when(u + NBUF < PER_W)
                def _():
                    pltpu.async_copy(
                        tab_hbm.at[idx_v.at[u + NBUF]], gbuf[b], gsem[b]
                    )
            return carry

        lax.fori_loop(0, PER_W // NBUF, outer, 0)

        for b in range(NBUF):
            unit_out_copies(u0 + PER_W - NBUF + b, b, issue=False)

    return k(x2d, table)


def kernel(x, table):
    x2d = jnp.transpose(x).reshape(NUNITS, CHUNK).astype(jnp.int32)
    out5 = _sc_embed(x2d, table)
    return out5.transpose(2, 4, 0, 1, 3).reshape(B, H, D)


# speedup vs baseline: 1.5354x; 1.5354x over previous
import functools
import jax
import jax.numpy as jnp
from jax import lax
from jax.experimental import pallas as pl
from jax.experimental.pallas import tpu as pltpu
from jax.experimental.pallas import tpu_sc as plsc

D = 64
B = 16384
H = 50
NC, NS = 2, 16
NW = NC * NS
CHUNK = 128
NUNITS = B * H // CHUNK
PER_W = NUNITS // NW
NBUF = 2
SCALE = 8.0
EG = D // 8


def _sc_embed(x2d, table):
    mesh = plsc.VectorSubcoreMesh(core_axis_name="c", subcore_axis_name="s")

    @functools.partial(
        pl.kernel,
        out_type=jax.ShapeDtypeStruct((H, EG, B // CHUNK, 8, CHUNK),
                                      jnp.float32),
        mesh=mesh,
        scratch_types=[
            pltpu.VMEM((PER_W, CHUNK), jnp.int32),
            pltpu.VMEM((CHUNK, D), jnp.float32),
            pltpu.VMEM((CHUNK, D), jnp.float32),
            pltpu.VMEM((D // 16, CHUNK, 16), jnp.float32),
            pltpu.VMEM((EG, 8, CHUNK), jnp.float32),
            pltpu.VMEM((EG, 8, CHUNK), jnp.float32),
            pltpu.SemaphoreType.DMA,
            pltpu.SemaphoreType.DMA,
            pltpu.SemaphoreType.DMA,
            pltpu.SemaphoreType.DMA,
        ],
        compiler_params=pltpu.CompilerParams(
            use_tc_tiling_on_sc=False, needs_layout_passes=False
        ),
    )
    def k(x_hbm, tab_hbm, out_hbm, idx_v, g0, g1, gp, w0, w1,
          gs0, gs1, ws0, ws1):
        gbuf, wbuf = [g0, g1], [w0, w1]
        gsem, wsem = [gs0, gs1], [ws0, ws1]
        wid = lax.axis_index("s") * NC + lax.axis_index("c")
        u0 = wid * PER_W

        pltpu.sync_copy(x_hbm.at[pl.ds(u0, PER_W)], idx_v)

        for b in range(NBUF):
            pltpu.async_copy(tab_hbm.at[idx_v.at[b]], gbuf[b], gsem[b])

        iota = lax.iota(jnp.int32, 16)

        def transpose_scale(gb, wb):
            @plsc.parallel_loop(0, CHUNK // 8)
            def srow(i):
                for r in range(8):
                    for t in range(D // 16):
                        gp[t, i * 8 + r, :] = (
                            gb[i * 8 + r, pl.ds(t * 16, 16)] * SCALE
                        )

            @plsc.parallel_loop(0, CHUNK // 16)
            def gblock(g):
                rowv = iota + g * 16
                for eg in range(EG):
                    for es in range(8):
                        e = eg * 8 + es
                        colv = jnp.full((16,), e % 16, jnp.int32)
                        v = plsc.load_gather(gp.at[e // 16], [rowv, colv])
                        wb[eg, es, pl.ds(g * 16, 16)] = v

        def unit_out_copies(u, b, issue):
            h = u // (B // CHUNK)
            bg = lax.rem(u, B // CHUNK)
            for eg in range(EG):
                cp = pltpu.make_async_copy(
                    wbuf[b].at[eg], out_hbm.at[h, eg, bg], wsem[b]
                )
                if issue:
                    cp.start()
                else:
                    cp.wait()

        def outer(g, carry):
            for b in range(NBUF):
                u = g * NBUF + b
                pltpu.make_async_copy(
                    tab_hbm.at[idx_v.at[u]], gbuf[b], gsem[b]
                ).wait()
                @pl.when(u >= NBUF)
                def _():
                    unit_out_copies(u0 + u - NBUF, b, issue=False)

                transpose_scale(gbuf[b], wbuf[b])
                unit_out_copies(u0 + u, b, issue=True)

                @pl.when(u + NBUF < PER_W)
                def _():
                    pltpu.async_copy(
                        tab_hbm.at[idx_v.at[u + NBUF]], gbuf[b], gsem[b]
                    )
            return carry

        lax.fori_loop(0, PER_W // NBUF, outer, 0)

        for b in range(NBUF):
            unit_out_copies(u0 + PER_W - NBUF + b, b, issue=False)

    return k(x2d, table)


def kernel(x, table):
    x2d = jnp.transpose(x).reshape(NUNITS, CHUNK).astype(jnp.int32)
    out5 = _sc_embed(x2d, table)
    return out5.transpose(2, 4, 0, 1, 3).reshape(B, H, D)
